# baseline (speedup 1.0000x reference)
import jax
import jax.numpy as jnp
from jax.experimental import pallas as pl
from jax.experimental.pallas import tpu as pltpu

BM = 200
C1 = 16
C2 = 32
VMEM_LIMIT = 64 * 1024 * 1024


def _t(j):
    return j - 1 - (j > C1).astype(jnp.int32) - (j > C2).astype(jnp.int32)


def _fused_body(adja_ref, adjb_ref, x_ref, w1_ref, b1_ref, w2_ref, b2_ref,
                out_ref, s2_ref, cache_ref):
    m = pl.num_programs(0) // 2
    i = pl.program_id(0)
    j = i - m

    def _layer1(src_ref):
        acc = jnp.dot(src_ref[...], x_ref[...],
                      preferred_element_type=jnp.float32)
        h = jnp.dot(acc, w1_ref[...], preferred_element_type=jnp.float32)
        h = jnp.maximum(h + b1_ref[...], 0.0)
        s2_ref[pl.ds(i * BM, BM), :] = jnp.dot(
            h, w2_ref[...], preferred_element_type=jnp.float32)

    @pl.when((i < m) & (i % 2 == 0))
    def _phase1_even():
        _layer1(adja_ref)

    @pl.when((i < m) & (i % 2 == 1))
    def _phase1_odd():
        _layer1(adjb_ref)

    @pl.when(i == m - 3)
    def _cache0():
        cache_ref[0] = adjb_ref[...]

    @pl.when(i == m - 2)
    def _cache1():
        cache_ref[1] = adja_ref[...]

    def _emit(src):
        t2 = jnp.dot(src, s2_ref[...], preferred_element_type=jnp.float32)
        out_ref[...] = jnp.max(t2 + b2_ref[...], axis=0, keepdims=True)[None]

    t = _t(j)
    is_fetch = (j >= 1) & (j != C1) & (j != C2)

    @pl.when(j == 0)
    def _phase2_live():
        _emit(adjb_ref[...])

    @pl.when(j == C1)
    def _phase2_c0():
        _emit(cache_ref[0])

    @pl.when(j == C2)
    def _phase2_c1():
        _emit(cache_ref[1])

    @pl.when(is_fetch & (t % 2 == 0))
    def _phase2_a():
        _emit(adja_ref[...])

    @pl.when(is_fetch & (t % 2 == 1))
    def _phase2_b():
        _emit(adjb_ref[...])


def _head_body(pm_ref, w3_ref, b3_ref, w4_ref, b4_ref, w5_ref, b5_ref, out_ref):
    v = jnp.max(pm_ref[...], axis=(0, 1), keepdims=False)[None]
    v = jnp.maximum(jnp.dot(v, w3_ref[...], preferred_element_type=jnp.float32)
                    + b3_ref[...], 0.0)
    v = jnp.maximum(jnp.dot(v, w4_ref[...], preferred_element_type=jnp.float32)
                    + b4_ref[...], 0.0)
    out_ref[...] = (jnp.dot(v, w5_ref[...], preferred_element_type=jnp.float32)
                    + b5_ref[...])


def kernel(x, adj, W1, b1, W2, b2, W3, b3, W4, b4, W5, b5):
    n, nfeat = x.shape
    nhid = W1.shape[1]
    n2 = W2.shape[1]
    ncls = W5.shape[1]
    m = n // BM

    def a_idx(i):
        j = i - m
        t = _t(j)
        p2 = jnp.where(t >= 0, t - t % 2, m - 2)
        return jnp.where(i < m, 2 * (i // 2), p2), 0

    def b_idx(i):
        j = i - m
        t = _t(j)
        p2 = jnp.where(t >= 1, t - 1 + t % 2, m - 1)
        return jnp.where(i < m, 2 * (i // 2) + 1, p2), 0

    def out_idx(i):
        j = i - m
        t = _t(j)
        row = jnp.where(j <= 0, m - 1,
                        jnp.where(j == C1, m - 3,
                                  jnp.where(j == C2, m - 2, t)))
        return row, 0, 0

    part_max = pl.pallas_call(
        _fused_body,
        grid=(2 * m,),
        in_specs=[
            pl.BlockSpec((BM, n), a_idx),
            pl.BlockSpec((BM, n), b_idx),
            pl.BlockSpec((n, nfeat), lambda i: (0, 0)),
            pl.BlockSpec((nfeat, nhid), lambda i: (0, 0)),
            pl.BlockSpec((1, nhid), lambda i: (0, 0)),
            pl.BlockSpec((nhid, n2), lambda i: (0, 0)),
            pl.BlockSpec((1, n2), lambda i: (0, 0)),
        ],
        out_specs=pl.BlockSpec((1, 1, n2), out_idx),
        out_shape=jax.ShapeDtypeStruct((m, 1, n2), jnp.float32),
        scratch_shapes=[
            pltpu.VMEM((n, n2), jnp.float32),
            pltpu.VMEM((2, BM, n), jnp.float32),
        ],
        compiler_params=pltpu.CompilerParams(
            dimension_semantics=("arbitrary",),
            vmem_limit_bytes=VMEM_LIMIT),
    )(adj, adj, x, W1, b1.reshape(1, -1), W2, b2.reshape(1, -1))

    out = pl.pallas_call(
        _head_body,
        in_specs=[
            pl.BlockSpec(part_max.shape, lambda: (0, 0, 0)),
            pl.BlockSpec(W3.shape, lambda: (0, 0)),
            pl.BlockSpec((1, W3.shape[1]), lambda: (0, 0)),
            pl.BlockSpec(W4.shape, lambda: (0, 0)),
            pl.BlockSpec((1, W4.shape[1]), lambda: (0, 0)),
            pl.BlockSpec(W5.shape, lambda: (0, 0)),
            pl.BlockSpec((1, ncls), lambda: (0, 0)),
        ],
        out_specs=pl.BlockSpec((1, ncls), lambda: (0, 0)),
        out_shape=jax.ShapeDtypeStruct((1, ncls), jnp.float32),
    )(part_max, W3, b3.reshape(1, -1), W4, b4.reshape(1, -1),
      W5, b5.reshape(1, -1))

    return out.reshape(ncls)


# speedup vs baseline: 1.2620x; 1.2620x over previous
import jax
import jax.numpy as jnp
from jax.experimental import pallas as pl
from jax.experimental.pallas import tpu as pltpu

BM = 400
C1 = 12
VMEM_LIMIT = 64 * 1024 * 1024


def _fused_body(adj_ref, x_ref, w1_ref, b1_ref, w2_ref, b2_ref,
                out_ref, s2_ref, cache_ref):
    m = pl.num_programs(0) // 2
    i = pl.program_id(0)
    j = i - m

    @pl.when(i < m)
    def _phase1():
        acc = jnp.dot(adj_ref[...], x_ref[...],
                      preferred_element_type=jnp.float32)
        h = jnp.dot(acc, w1_ref[...], preferred_element_type=jnp.float32)
        h = jnp.maximum(h + b1_ref[...], 0.0)
        s2_ref[pl.ds(i * BM, BM), :] = jnp.dot(
            h, w2_ref[...], preferred_element_type=jnp.float32)

        @pl.when(i == m - 2)
        def _():
            cache_ref[0] = adj_ref[...]

    def _emit(src):
        t2 = jnp.dot(src, s2_ref[...], preferred_element_type=jnp.float32)
        out_ref[...] = jnp.max(t2 + b2_ref[...], axis=0, keepdims=True)[None]

    @pl.when((j == 0) | ((j >= 1) & (j != C1)))
    def _phase2_streamed():
        _emit(adj_ref[...])

    @pl.when(j == C1)
    def _phase2_cached():
        _emit(cache_ref[0])


def _head_body(pm_ref, w3_ref, b3_ref, w4_ref, b4_ref, w5_ref, b5_ref, out_ref):
    v = jnp.max(pm_ref[...], axis=(0, 1), keepdims=False)[None]
    v = jnp.maximum(jnp.dot(v, w3_ref[...], preferred_element_type=jnp.float32)
                    + b3_ref[...], 0.0)
    v = jnp.maximum(jnp.dot(v, w4_ref[...], preferred_element_type=jnp.float32)
                    + b4_ref[...], 0.0)
    out_ref[...] = (jnp.dot(v, w5_ref[...], preferred_element_type=jnp.float32)
                    + b5_ref[...])


def kernel(x, adj, W1, b1, W2, b2, W3, b3, W4, b4, W5, b5):
    n, nfeat = x.shape
    nhid = W1.shape[1]
    n2 = W2.shape[1]
    ncls = W5.shape[1]
    m = n // BM

    def adj_idx(i):
        j = i - m
        t = j - 1 - (j > C1).astype(jnp.int32)
        p2 = jnp.where(j <= 0, m - 1, jnp.where(j == C1, t - 1, t))
        return jnp.where(i < m, i, p2), 0

    def out_idx(i):
        j = i - m
        t = j - 1 - (j > C1).astype(jnp.int32)
        row = jnp.where(j <= 0, m - 1, jnp.where(j == C1, m - 2, t))
        return row, 0, 0

    part_max = pl.pallas_call(
        _fused_body,
        grid=(2 * m,),
        in_specs=[
            pl.BlockSpec((BM, n), adj_idx),
            pl.BlockSpec((n, nfeat), lambda i: (0, 0)),
            pl.BlockSpec((nfeat, nhid), lambda i: (0, 0)),
            pl.BlockSpec((1, nhid), lambda i: (0, 0)),
            pl.BlockSpec((nhid, n2), lambda i: (0, 0)),
            pl.BlockSpec((1, n2), lambda i: (0, 0)),
        ],
        out_specs=pl.BlockSpec((1, 1, n2), out_idx),
        out_shape=jax.ShapeDtypeStruct((m, 1, n2), jnp.float32),
        scratch_shapes=[
            pltpu.VMEM((n, n2), jnp.float32),
            pltpu.VMEM((1, BM, n), jnp.float32),
        ],
        compiler_params=pltpu.CompilerParams(
            dimension_semantics=("arbitrary",),
            vmem_limit_bytes=VMEM_LIMIT),
    )(adj, x, W1, b1.reshape(1, -1), W2, b2.reshape(1, -1))

    out = pl.pallas_call(
        _head_body,
        in_specs=[
            pl.BlockSpec(part_max.shape, lambda: (0, 0, 0)),
            pl.BlockSpec(W3.shape, lambda: (0, 0)),
            pl.BlockSpec((1, W3.shape[1]), lambda: (0, 0)),
            pl.BlockSpec(W4.shape, lambda: (0, 0)),
            pl.BlockSpec((1, W4.shape[1]), lambda: (0, 0)),
            pl.BlockSpec(W5.shape, lambda: (0, 0)),
            pl.BlockSpec((1, ncls), lambda: (0, 0)),
        ],
        out_specs=pl.BlockSpec((1, ncls), lambda: (0, 0)),
        out_shape=jax.ShapeDtypeStruct((1, ncls), jnp.float32),
    )(part_max, W3, b3.reshape(1, -1), W4, b4.reshape(1, -1),
      W5, b5.reshape(1, -1))

    return out.reshape(ncls)
